# baseline (speedup 1.0000x reference)
import functools

import jax
import jax.numpy as jnp
from jax import lax
from jax.experimental import pallas as pl
from jax.experimental.pallas import tpu as pltpu
from jax.experimental.pallas import tpu_sc as plsc

N = 8192
D = 1024
NC = 2
NS = 16
NW = NC * NS
R = N // NW
C = 64
NCH = R // C


def _make_copy():
    mesh = plsc.VectorSubcoreMesh(core_axis_name="c", subcore_axis_name="s")
    scratch = [pltpu.VMEM_SHARED((NS, 2, C, D), jnp.float32)]
    scratch += [pltpu.SemaphoreType.DMA for _ in range(4)]

    @functools.partial(
        pl.kernel,
        mesh=mesh,
        out_type=jax.ShapeDtypeStruct((N, D), jnp.float32),
        scratch_types=scratch,
    )
    def copy_kernel(table_hbm, out_hbm, shared, *sems):
        gsem = sems[:2]
        ssem = sems[2:]
        wid = lax.axis_index("s") * NC + lax.axis_index("c")
        sid = lax.axis_index("s")
        row0 = wid * R

        def gather(g, b):
            return pltpu.make_async_copy(
                table_hbm.at[pl.ds(row0 + g * C, C)],
                shared.at[sid].at[b], gsem[b])

        def store(g, b):
            return pltpu.make_async_copy(
                shared.at[sid].at[b],
                out_hbm.at[pl.ds(row0 + g * C, C)], ssem[b])

        gathers = [None] * NCH
        stores = [None] * NCH
        for g in range(2):
            gathers[g] = gather(g, g)
            gathers[g].start()
        for g in range(NCH):
            b = g % 2
            gathers[g].wait()
            stores[g] = store(g, b)
            stores[g].start()
            if g + 2 < NCH:
                stores[g].wait()
                gathers[g + 2] = gather(g + 2, b)
                gathers[g + 2].start()
        for g in range(NCH - 2, NCH):
            stores[g].wait()

    return copy_kernel


_copy = _make_copy()


@jax.jit
def kernel(seq_len, table):
    del seq_len
    return _copy(table)


# speedup vs baseline: 1.0165x; 1.0165x over previous
import functools

import jax
import jax.numpy as jnp
from jax import lax
from jax.experimental import pallas as pl
from jax.experimental.pallas import tpu as pltpu
from jax.experimental.pallas import tpu_sc as plsc

N = 8192
D = 1024
NC = 2
NS = 16
NW = NC * NS
R = N // NW

PLAN = [(0, 64), (64, 56), (120, 64), (184, 56), (240, 16)]
BUFROWS = [64, 56]
NCH = len(PLAN)


def _make_gather():
    mesh = plsc.VectorSubcoreMesh(core_axis_name="c", subcore_axis_name="s")
    scratch = [pltpu.VMEM((R,), jnp.int32)]
    scratch += [pltpu.VMEM((br, D), jnp.float32) for br in BUFROWS]
    scratch += [pltpu.SemaphoreType.DMA for _ in range(4)]

    @functools.partial(
        pl.kernel,
        mesh=mesh,
        out_type=jax.ShapeDtypeStruct((N, D), jnp.float32),
        scratch_types=scratch,
    )
    def gather_kernel(table_hbm, idx_hbm, out_hbm, idx_v, *rest):
        bufs = rest[:2]
        gsem = rest[2:4]
        ssem = rest[4:]
        wid = lax.axis_index("s") * NC + lax.axis_index("c")
        row0 = wid * R

        pltpu.sync_copy(idx_hbm.at[pl.ds(row0, R)], idx_v)

        def buf_at(b, ln):
            return bufs[b] if ln == BUFROWS[b] else bufs[b].at[pl.ds(0, ln)]

        def gather(g):
            off, ln = PLAN[g]
            b = g % 2
            return pltpu.make_async_copy(
                table_hbm.at[idx_v.at[pl.ds(off, ln)]], buf_at(b, ln),
                gsem[b])

        def store(g):
            off, ln = PLAN[g]
            b = g % 2
            return pltpu.make_async_copy(
                buf_at(b, ln), out_hbm.at[pl.ds(row0 + off, ln)], ssem[b])

        gathers = [None] * NCH
        stores = [None] * NCH
        for g in range(2):
            gathers[g] = gather(g)
            gathers[g].start()
        for g in range(NCH):
            gathers[g].wait()
            stores[g] = store(g)
            stores[g].start()
            if g + 2 < NCH:
                stores[g].wait()
                gathers[g + 2] = gather(g + 2)
                gathers[g + 2].start()
        for g in range(NCH - 2, NCH):
            stores[g].wait()

    return gather_kernel


_gather = _make_gather()


@jax.jit
def kernel(seq_len, table):
    n, _ = table.shape
    shift = jnp.asarray(seq_len, jnp.int32) - n
    idx = jnp.clip(jnp.arange(n, dtype=jnp.int32) + shift, 0, n - 1)
    return _gather(table, idx)
